# baseline (speedup 1.0000x reference)
import jax
import jax.numpy as jnp
from jax import lax
from jax.experimental import pallas as pl
from jax.experimental.pallas import tpu as pltpu
from jax.experimental.pallas import tpu_sc as plsc

_N = 2 * 96 * 224 * 224
_NW = 32
_CH = 21504
_NCH = 6
_PER_W = _CH * _NCH
_N_SC = _NW * _PER_W
_N_TC = _N - _N_SC
_L = 16
_UN = 8

_THRESH = [0.75, 0.375, 0.1875, 0.09375, 0.046875, 0.0234375, 0.01171875]
_VALS = [1.0, 0.5, 0.25, 0.125, 0.0625, 0.03125, 0.015625, 0.0078125]


def _quant_vec(v):
    a = jnp.abs(v)
    mag = jnp.full_like(a, _VALS[7])
    for t, val in zip(reversed(_THRESH), reversed(_VALS[:7])):
        mag = jnp.where(a >= t, val, mag)
    neg = v <= 0.0
    return jnp.where(neg, -mag, mag)


def _compute_chunk(in_b, out_b):
    def fb(j, c):
        o = j * (_L * _UN)
        for u in range(_UN):
            s = pl.ds(o + u * _L, _L)
            out_b[s] = _quant_vec(in_b[s])
        return c

    lax.fori_loop(0, _CH // (_L * _UN), fb, jnp.int32(0))


def _sc_body(x_hbm, o_hbm, in0, in1, out0, out1, si0, si1, so0, so1):
    wid = lax.axis_index("s") * 2 + lax.axis_index("c")
    base = wid * _PER_W
    bufs_in = (in0, in1)
    bufs_out = (out0, out1)
    sems_in = (si0, si1)
    sems_out = (so0, so1)
    in_h = [None, None]
    out_h = [None, None]
    in_h[0] = pltpu.async_copy(x_hbm.at[pl.ds(base, _CH)], bufs_in[0],
                               sems_in[0])
    for i in range(_NCH):
        b = i % 2
        nb = (i + 1) % 2
        if i + 1 < _NCH:
            in_h[nb] = pltpu.async_copy(
                x_hbm.at[pl.ds(base + (i + 1) * _CH, _CH)], bufs_in[nb],
                sems_in[nb])
        in_h[b].wait()
        if i >= 2:
            out_h[b].wait()
        _compute_chunk(bufs_in[b], bufs_out[b])
        out_h[b] = pltpu.async_copy(bufs_out[b],
                                    o_hbm.at[pl.ds(base + i * _CH, _CH)],
                                    sems_out[b])
    out_h[(_NCH - 2) % 2].wait()
    out_h[(_NCH - 1) % 2].wait()


def _sc_kernel(xf):
    mesh = plsc.VectorSubcoreMesh(core_axis_name="c", subcore_axis_name="s")
    run = pl.kernel(
        _sc_body,
        out_type=jax.ShapeDtypeStruct((_N_SC,), jnp.float32),
        mesh=mesh,
        scratch_types=[
            pltpu.VMEM((_CH,), jnp.float32), pltpu.VMEM((_CH,), jnp.float32),
            pltpu.VMEM((_CH,), jnp.float32), pltpu.VMEM((_CH,), jnp.float32),
            pltpu.SemaphoreType.DMA, pltpu.SemaphoreType.DMA,
            pltpu.SemaphoreType.DMA, pltpu.SemaphoreType.DMA,
        ],
    )
    return run(xf)


def _quant_tc_block(x):
    a = jnp.clip(jnp.abs(x), 0.0078125, 1.0)
    bits = lax.bitcast_convert_type(a, jnp.int32)
    neg = x <= 0.0
    add = jnp.where(neg, jnp.int32(0x400000), jnp.int32(0x3FFFFF))
    pb = (bits + add) & jnp.int32(0x7F800000)
    mag = lax.bitcast_convert_type(pb, jnp.float32)
    return jnp.where(neg, -mag, mag)


def _tc_body(x_ref, o_ref):
    o_ref[...] = _quant_tc_block(x_ref[...])


_COLS = 1024
_TC_ROWS = _N_TC // _COLS
_SC_ROWS = _N_SC // _COLS
_TC_BLK = 224
_TC_GRID = _TC_ROWS // _TC_BLK
_SC_ROW_OFF = _SC_ROWS // _TC_BLK


def _tc_kernel(xf2d):
    return pl.pallas_call(
        _tc_body,
        out_shape=jax.ShapeDtypeStruct((_TC_ROWS, _COLS), jnp.float32),
        grid=(_TC_GRID,),
        in_specs=[pl.BlockSpec((_TC_BLK, _COLS),
                               lambda i: (i + _SC_ROW_OFF, 0))],
        out_specs=pl.BlockSpec((_TC_BLK, _COLS), lambda i: (i, 0)),
    )(xf2d)


def kernel(x, pow2_values):
    B, C, W, H = x.shape
    xf = x.reshape(_N)
    out_sc = _sc_kernel(xf)
    out_tc = _tc_kernel(xf.reshape(_N // _COLS, _COLS))
    out = jnp.concatenate([out_sc, out_tc.reshape(_N_TC)])
    return out.reshape(B, C, W, H)


# speedup vs baseline: 3.6974x; 3.6974x over previous
import jax
import jax.numpy as jnp
from jax import lax
from jax.experimental import pallas as pl
from jax.experimental.pallas import tpu as pltpu
from jax.experimental.pallas import tpu_sc as plsc

_N = 2 * 96 * 224 * 224
_NW = 32
_CH = 21504
_NCH = 6
_PER_W = _CH * _NCH
_N_SC = _NW * _PER_W
_N_TC = _N - _N_SC
_L = 16
_UN = 8

_THRESH = [0.75, 0.375, 0.1875, 0.09375, 0.046875, 0.0234375, 0.01171875]
_VALS = [1.0, 0.5, 0.25, 0.125, 0.0625, 0.03125, 0.015625, 0.0078125]


def _quant_vec(v):
    a = jnp.abs(v)
    mag = jnp.full_like(a, _VALS[7])
    for t, val in zip(reversed(_THRESH), reversed(_VALS[:7])):
        mag = jnp.where(a >= t, val, mag)
    neg = v <= 0.0
    return jnp.where(neg, -mag, mag)


def _compute_chunk(in_b, out_b):
    def fb(j, c):
        o = j * (_L * _UN)
        for u in range(_UN):
            s = pl.ds(o + u * _L, _L)
            out_b[s] = _quant_vec(in_b[s])
        return c

    lax.fori_loop(0, _CH // (_L * _UN), fb, jnp.int32(0))


def _sc_body(x_hbm, o_hbm, in0, in1, out0, out1, si0, si1, so0, so1):
    wid = lax.axis_index("s") * 2 + lax.axis_index("c")
    base = wid * _PER_W
    bufs_in = (in0, in1)
    bufs_out = (out0, out1)
    sems_in = (si0, si1)
    sems_out = (so0, so1)
    in_h = [None, None]
    out_h = [None, None]
    in_h[0] = pltpu.async_copy(x_hbm.at[pl.ds(base, _CH)], bufs_in[0],
                               sems_in[0])
    for i in range(_NCH):
        b = i % 2
        nb = (i + 1) % 2
        if i + 1 < _NCH:
            in_h[nb] = pltpu.async_copy(
                x_hbm.at[pl.ds(base + (i + 1) * _CH, _CH)], bufs_in[nb],
                sems_in[nb])
        in_h[b].wait()
        if i >= 2:
            out_h[b].wait()
        _compute_chunk(bufs_in[b], bufs_out[b])
        out_h[b] = pltpu.async_copy(bufs_out[b],
                                    o_hbm.at[pl.ds(base + i * _CH, _CH)],
                                    sems_out[b])
    out_h[(_NCH - 2) % 2].wait()
    out_h[(_NCH - 1) % 2].wait()


def _sc_kernel(xf):
    mesh = plsc.VectorSubcoreMesh(core_axis_name="c", subcore_axis_name="s")
    run = pl.kernel(
        _sc_body,
        out_type=jax.ShapeDtypeStruct((_N_SC,), jnp.float32),
        mesh=mesh,
        scratch_types=[
            pltpu.VMEM((_CH,), jnp.float32), pltpu.VMEM((_CH,), jnp.float32),
            pltpu.VMEM((_CH,), jnp.float32), pltpu.VMEM((_CH,), jnp.float32),
            pltpu.SemaphoreType.DMA, pltpu.SemaphoreType.DMA,
            pltpu.SemaphoreType.DMA, pltpu.SemaphoreType.DMA,
        ],
    )
    return run(xf)


def _quant_tc_block(x):
    a = jnp.clip(jnp.abs(x), 0.0078125, 1.0)
    bits = lax.bitcast_convert_type(a, jnp.int32)
    neg = x <= 0.0
    add = jnp.where(neg, jnp.int32(0x400000), jnp.int32(0x3FFFFF))
    pb = (bits + add) & jnp.int32(0x7F800000)
    mag = lax.bitcast_convert_type(pb, jnp.float32)
    return jnp.where(neg, -mag, mag)


def _tc_body(x_ref, o_ref):
    o_ref[...] = _quant_tc_block(x_ref[...])


_COLS = 1024
_TC_ROWS = _N_TC // _COLS
_SC_ROWS = _N_SC // _COLS
_TC_BLK = 224
_TC_GRID = _TC_ROWS // _TC_BLK
_SC_ROW_OFF = _SC_ROWS // _TC_BLK


def _tc_kernel(xf2d):
    return pl.pallas_call(
        _tc_body,
        out_shape=jax.ShapeDtypeStruct((_TC_ROWS, _COLS), jnp.float32),
        grid=(_TC_GRID,),
        in_specs=[pl.BlockSpec((_TC_BLK, _COLS),
                               lambda i: (i + _SC_ROW_OFF, 0))],
        out_specs=pl.BlockSpec((_TC_BLK, _COLS), lambda i: (i, 0)),
    )(xf2d)


_NAT_ROWS = 2 * 96 * 224
_NAT_COLS = 224
_NAT_BLK = 448
_NAT_GRID = _NAT_ROWS // _NAT_BLK


def _tc_native(x3):
    return pl.pallas_call(
        _tc_body,
        out_shape=jax.ShapeDtypeStruct((_NAT_ROWS, _NAT_COLS), jnp.float32),
        grid=(_NAT_GRID,),
        in_specs=[pl.BlockSpec((_NAT_BLK, _NAT_COLS), lambda i: (i, 0))],
        out_specs=pl.BlockSpec((_NAT_BLK, _NAT_COLS), lambda i: (i, 0)),
    )(x3)


def kernel(x, pow2_values):
    B, C, W, H = x.shape
    out = _tc_native(x.reshape(_NAT_ROWS, _NAT_COLS))
    return out.reshape(B, C, W, H)


# speedup vs baseline: 5.1524x; 1.3935x over previous
import jax
import jax.numpy as jnp
from jax import lax
from jax.experimental import pallas as pl
from jax.experimental.pallas import tpu as pltpu
from jax.experimental.pallas import tpu_sc as plsc

_N = 2 * 96 * 224 * 224
_NW = 32
_CH = 21504
_NCH = 6
_PER_W = _CH * _NCH
_N_SC = _NW * _PER_W
_N_TC = _N - _N_SC
_L = 16
_UN = 8

_THRESH = [0.75, 0.375, 0.1875, 0.09375, 0.046875, 0.0234375, 0.01171875]
_VALS = [1.0, 0.5, 0.25, 0.125, 0.0625, 0.03125, 0.015625, 0.0078125]


def _quant_vec(v):
    a = jnp.abs(v)
    mag = jnp.full_like(a, _VALS[7])
    for t, val in zip(reversed(_THRESH), reversed(_VALS[:7])):
        mag = jnp.where(a >= t, val, mag)
    neg = v <= 0.0
    return jnp.where(neg, -mag, mag)


def _compute_chunk(in_b, out_b):
    def fb(j, c):
        o = j * (_L * _UN)
        for u in range(_UN):
            s = pl.ds(o + u * _L, _L)
            out_b[s] = _quant_vec(in_b[s])
        return c

    lax.fori_loop(0, _CH // (_L * _UN), fb, jnp.int32(0))


def _sc_body(x_hbm, o_hbm, in0, in1, out0, out1, si0, si1, so0, so1):
    wid = lax.axis_index("s") * 2 + lax.axis_index("c")
    base = wid * _PER_W
    bufs_in = (in0, in1)
    bufs_out = (out0, out1)
    sems_in = (si0, si1)
    sems_out = (so0, so1)
    in_h = [None, None]
    out_h = [None, None]
    in_h[0] = pltpu.async_copy(x_hbm.at[pl.ds(base, _CH)], bufs_in[0],
                               sems_in[0])
    for i in range(_NCH):
        b = i % 2
        nb = (i + 1) % 2
        if i + 1 < _NCH:
            in_h[nb] = pltpu.async_copy(
                x_hbm.at[pl.ds(base + (i + 1) * _CH, _CH)], bufs_in[nb],
                sems_in[nb])
        in_h[b].wait()
        if i >= 2:
            out_h[b].wait()
        _compute_chunk(bufs_in[b], bufs_out[b])
        out_h[b] = pltpu.async_copy(bufs_out[b],
                                    o_hbm.at[pl.ds(base + i * _CH, _CH)],
                                    sems_out[b])
    out_h[(_NCH - 2) % 2].wait()
    out_h[(_NCH - 1) % 2].wait()


def _sc_kernel(xf):
    mesh = plsc.VectorSubcoreMesh(core_axis_name="c", subcore_axis_name="s")
    run = pl.kernel(
        _sc_body,
        out_type=jax.ShapeDtypeStruct((_N_SC,), jnp.float32),
        mesh=mesh,
        scratch_types=[
            pltpu.VMEM((_CH,), jnp.float32), pltpu.VMEM((_CH,), jnp.float32),
            pltpu.VMEM((_CH,), jnp.float32), pltpu.VMEM((_CH,), jnp.float32),
            pltpu.SemaphoreType.DMA, pltpu.SemaphoreType.DMA,
            pltpu.SemaphoreType.DMA, pltpu.SemaphoreType.DMA,
        ],
    )
    return run(xf)


def _quant_tc_block(x):
    a = jnp.clip(jnp.abs(x), 0.0078125, 1.0)
    bits = lax.bitcast_convert_type(a, jnp.int32)
    neg = x <= 0.0
    add = jnp.where(neg, jnp.int32(0x400000), jnp.int32(0x3FFFFF))
    pb = (bits + add) & jnp.int32(0x7F800000)
    mag = lax.bitcast_convert_type(pb, jnp.float32)
    return jnp.where(neg, -mag, mag)


def _tc_body(x_ref, o_ref):
    o_ref[...] = _quant_tc_block(x_ref[...])


_COLS = 1024
_TC_ROWS = _N_TC // _COLS
_SC_ROWS = _N_SC // _COLS
_TC_BLK = 224
_TC_GRID = _TC_ROWS // _TC_BLK
_SC_ROW_OFF = _SC_ROWS // _TC_BLK


def _tc_kernel(xf2d):
    return pl.pallas_call(
        _tc_body,
        out_shape=jax.ShapeDtypeStruct((_TC_ROWS, _COLS), jnp.float32),
        grid=(_TC_GRID,),
        in_specs=[pl.BlockSpec((_TC_BLK, _COLS),
                               lambda i: (i + _SC_ROW_OFF, 0))],
        out_specs=pl.BlockSpec((_TC_BLK, _COLS), lambda i: (i, 0)),
    )(xf2d)


_NAT_ROWS = 2 * 96 * 224
_NAT_COLS = 224
_NAT_BLK = 896
_NAT_GRID = _NAT_ROWS // _NAT_BLK


def _tc_native(x3):
    return pl.pallas_call(
        _tc_body,
        out_shape=jax.ShapeDtypeStruct((_NAT_ROWS, _NAT_COLS), jnp.float32),
        grid=(_NAT_GRID,),
        in_specs=[pl.BlockSpec((_NAT_BLK, _NAT_COLS), lambda i: (i, 0))],
        out_specs=pl.BlockSpec((_NAT_BLK, _NAT_COLS), lambda i: (i, 0)),
    )(x3)


def kernel(x, pow2_values):
    B, C, W, H = x.shape
    out = _tc_native(x.reshape(_NAT_ROWS, _NAT_COLS))
    return out.reshape(B, C, W, H)


# speedup vs baseline: 7.2512x; 1.4073x over previous
import jax
import jax.numpy as jnp
from jax import lax
from jax.experimental import pallas as pl
from jax.experimental.pallas import tpu as pltpu
from jax.experimental.pallas import tpu_sc as plsc

_N = 2 * 96 * 224 * 224
_NW = 32
_CH = 21504
_NCH = 6
_PER_W = _CH * _NCH
_N_SC = _NW * _PER_W
_N_TC = _N - _N_SC
_L = 16
_UN = 8

_THRESH = [0.75, 0.375, 0.1875, 0.09375, 0.046875, 0.0234375, 0.01171875]
_VALS = [1.0, 0.5, 0.25, 0.125, 0.0625, 0.03125, 0.015625, 0.0078125]


def _quant_vec(v):
    a = jnp.abs(v)
    mag = jnp.full_like(a, _VALS[7])
    for t, val in zip(reversed(_THRESH), reversed(_VALS[:7])):
        mag = jnp.where(a >= t, val, mag)
    neg = v <= 0.0
    return jnp.where(neg, -mag, mag)


def _compute_chunk(in_b, out_b):
    def fb(j, c):
        o = j * (_L * _UN)
        for u in range(_UN):
            s = pl.ds(o + u * _L, _L)
            out_b[s] = _quant_vec(in_b[s])
        return c

    lax.fori_loop(0, _CH // (_L * _UN), fb, jnp.int32(0))


def _sc_body(x_hbm, o_hbm, in0, in1, out0, out1, si0, si1, so0, so1):
    wid = lax.axis_index("s") * 2 + lax.axis_index("c")
    base = wid * _PER_W
    bufs_in = (in0, in1)
    bufs_out = (out0, out1)
    sems_in = (si0, si1)
    sems_out = (so0, so1)
    in_h = [None, None]
    out_h = [None, None]
    in_h[0] = pltpu.async_copy(x_hbm.at[pl.ds(base, _CH)], bufs_in[0],
                               sems_in[0])
    for i in range(_NCH):
        b = i % 2
        nb = (i + 1) % 2
        if i + 1 < _NCH:
            in_h[nb] = pltpu.async_copy(
                x_hbm.at[pl.ds(base + (i + 1) * _CH, _CH)], bufs_in[nb],
                sems_in[nb])
        in_h[b].wait()
        if i >= 2:
            out_h[b].wait()
        _compute_chunk(bufs_in[b], bufs_out[b])
        out_h[b] = pltpu.async_copy(bufs_out[b],
                                    o_hbm.at[pl.ds(base + i * _CH, _CH)],
                                    sems_out[b])
    out_h[(_NCH - 2) % 2].wait()
    out_h[(_NCH - 1) % 2].wait()


def _sc_kernel(xf):
    mesh = plsc.VectorSubcoreMesh(core_axis_name="c", subcore_axis_name="s")
    run = pl.kernel(
        _sc_body,
        out_type=jax.ShapeDtypeStruct((_N_SC,), jnp.float32),
        mesh=mesh,
        scratch_types=[
            pltpu.VMEM((_CH,), jnp.float32), pltpu.VMEM((_CH,), jnp.float32),
            pltpu.VMEM((_CH,), jnp.float32), pltpu.VMEM((_CH,), jnp.float32),
            pltpu.SemaphoreType.DMA, pltpu.SemaphoreType.DMA,
            pltpu.SemaphoreType.DMA, pltpu.SemaphoreType.DMA,
        ],
    )
    return run(xf)


def _quant_tc_block(x):
    a = jnp.clip(jnp.abs(x), 0.0078125, 1.0)
    bits = lax.bitcast_convert_type(a, jnp.int32)
    neg = x <= 0.0
    add = jnp.where(neg, jnp.int32(0x400000), jnp.int32(0x3FFFFF))
    pb = (bits + add) & jnp.int32(0x7F800000)
    mag = lax.bitcast_convert_type(pb, jnp.float32)
    return jnp.where(neg, -mag, mag)


def _tc_body(x_ref, o_ref):
    o_ref[...] = _quant_tc_block(x_ref[...])


_COLS = 1024
_TC_ROWS = _N_TC // _COLS
_SC_ROWS = _N_SC // _COLS
_TC_BLK = 224
_TC_GRID = _TC_ROWS // _TC_BLK
_SC_ROW_OFF = _SC_ROWS // _TC_BLK


def _tc_kernel(xf2d):
    return pl.pallas_call(
        _tc_body,
        out_shape=jax.ShapeDtypeStruct((_TC_ROWS, _COLS), jnp.float32),
        grid=(_TC_GRID,),
        in_specs=[pl.BlockSpec((_TC_BLK, _COLS),
                               lambda i: (i + _SC_ROW_OFF, 0))],
        out_specs=pl.BlockSpec((_TC_BLK, _COLS), lambda i: (i, 0)),
    )(xf2d)


_NAT_ROWS = 2 * 96 * 224
_NAT_COLS = 224
_NAT_BLK = 1792
_NAT_GRID = _NAT_ROWS // _NAT_BLK


def _tc_native(x3):
    return pl.pallas_call(
        _tc_body,
        out_shape=jax.ShapeDtypeStruct((_NAT_ROWS, _NAT_COLS), jnp.float32),
        grid=(_NAT_GRID,),
        in_specs=[pl.BlockSpec((_NAT_BLK, _NAT_COLS), lambda i: (i, 0))],
        out_specs=pl.BlockSpec((_NAT_BLK, _NAT_COLS), lambda i: (i, 0)),
    )(x3)


def kernel(x, pow2_values):
    B, C, W, H = x.shape
    out = _tc_native(x.reshape(_NAT_ROWS, _NAT_COLS))
    return out.reshape(B, C, W, H)


# speedup vs baseline: 8.4591x; 1.1666x over previous
import jax
import jax.numpy as jnp
from jax import lax
from jax.experimental import pallas as pl
from jax.experimental.pallas import tpu as pltpu
from jax.experimental.pallas import tpu_sc as plsc

_N = 2 * 96 * 224 * 224
_NW = 32
_CH = 21504
_NCH = 6
_PER_W = _CH * _NCH
_N_SC = _NW * _PER_W
_N_TC = _N - _N_SC
_L = 16
_UN = 8

_THRESH = [0.75, 0.375, 0.1875, 0.09375, 0.046875, 0.0234375, 0.01171875]
_VALS = [1.0, 0.5, 0.25, 0.125, 0.0625, 0.03125, 0.015625, 0.0078125]


def _quant_vec(v):
    a = jnp.abs(v)
    mag = jnp.full_like(a, _VALS[7])
    for t, val in zip(reversed(_THRESH), reversed(_VALS[:7])):
        mag = jnp.where(a >= t, val, mag)
    neg = v <= 0.0
    return jnp.where(neg, -mag, mag)


def _compute_chunk(in_b, out_b):
    def fb(j, c):
        o = j * (_L * _UN)
        for u in range(_UN):
            s = pl.ds(o + u * _L, _L)
            out_b[s] = _quant_vec(in_b[s])
        return c

    lax.fori_loop(0, _CH // (_L * _UN), fb, jnp.int32(0))


def _sc_body(x_hbm, o_hbm, in0, in1, out0, out1, si0, si1, so0, so1):
    wid = lax.axis_index("s") * 2 + lax.axis_index("c")
    base = wid * _PER_W
    bufs_in = (in0, in1)
    bufs_out = (out0, out1)
    sems_in = (si0, si1)
    sems_out = (so0, so1)
    in_h = [None, None]
    out_h = [None, None]
    in_h[0] = pltpu.async_copy(x_hbm.at[pl.ds(base, _CH)], bufs_in[0],
                               sems_in[0])
    for i in range(_NCH):
        b = i % 2
        nb = (i + 1) % 2
        if i + 1 < _NCH:
            in_h[nb] = pltpu.async_copy(
                x_hbm.at[pl.ds(base + (i + 1) * _CH, _CH)], bufs_in[nb],
                sems_in[nb])
        in_h[b].wait()
        if i >= 2:
            out_h[b].wait()
        _compute_chunk(bufs_in[b], bufs_out[b])
        out_h[b] = pltpu.async_copy(bufs_out[b],
                                    o_hbm.at[pl.ds(base + i * _CH, _CH)],
                                    sems_out[b])
    out_h[(_NCH - 2) % 2].wait()
    out_h[(_NCH - 1) % 2].wait()


def _sc_kernel(xf):
    mesh = plsc.VectorSubcoreMesh(core_axis_name="c", subcore_axis_name="s")
    run = pl.kernel(
        _sc_body,
        out_type=jax.ShapeDtypeStruct((_N_SC,), jnp.float32),
        mesh=mesh,
        scratch_types=[
            pltpu.VMEM((_CH,), jnp.float32), pltpu.VMEM((_CH,), jnp.float32),
            pltpu.VMEM((_CH,), jnp.float32), pltpu.VMEM((_CH,), jnp.float32),
            pltpu.SemaphoreType.DMA, pltpu.SemaphoreType.DMA,
            pltpu.SemaphoreType.DMA, pltpu.SemaphoreType.DMA,
        ],
    )
    return run(xf)


def _quant_tc_block(x):
    a = jnp.clip(jnp.abs(x), 0.0078125, 1.0)
    bits = lax.bitcast_convert_type(a, jnp.int32)
    neg = x <= 0.0
    add = jnp.where(neg, jnp.int32(0x400000), jnp.int32(0x3FFFFF))
    pb = (bits + add) & jnp.int32(0x7F800000)
    mag = lax.bitcast_convert_type(pb, jnp.float32)
    return jnp.where(neg, -mag, mag)


def _tc_body(x_ref, o_ref):
    o_ref[...] = _quant_tc_block(x_ref[...])


_COLS = 1024
_TC_ROWS = _N_TC // _COLS
_SC_ROWS = _N_SC // _COLS
_TC_BLK = 224
_TC_GRID = _TC_ROWS // _TC_BLK
_SC_ROW_OFF = _SC_ROWS // _TC_BLK


def _tc_kernel(xf2d):
    return pl.pallas_call(
        _tc_body,
        out_shape=jax.ShapeDtypeStruct((_TC_ROWS, _COLS), jnp.float32),
        grid=(_TC_GRID,),
        in_specs=[pl.BlockSpec((_TC_BLK, _COLS),
                               lambda i: (i + _SC_ROW_OFF, 0))],
        out_specs=pl.BlockSpec((_TC_BLK, _COLS), lambda i: (i, 0)),
    )(xf2d)


_NAT_ROWS = 2 * 96 * 224
_NAT_COLS = 224
_NAT_BLK = 3584
_NAT_GRID = _NAT_ROWS // _NAT_BLK


def _tc_native(x3):
    return pl.pallas_call(
        _tc_body,
        out_shape=jax.ShapeDtypeStruct((_NAT_ROWS, _NAT_COLS), jnp.float32),
        grid=(_NAT_GRID,),
        in_specs=[pl.BlockSpec((_NAT_BLK, _NAT_COLS), lambda i: (i, 0))],
        out_specs=pl.BlockSpec((_NAT_BLK, _NAT_COLS), lambda i: (i, 0)),
    )(x3)


def kernel(x, pow2_values):
    B, C, W, H = x.shape
    out = _tc_native(x.reshape(_NAT_ROWS, _NAT_COLS))
    return out.reshape(B, C, W, H)


# speedup vs baseline: 8.7210x; 1.0310x over previous
import jax
import jax.numpy as jnp
from jax import lax
from jax.experimental import pallas as pl
from jax.experimental.pallas import tpu as pltpu
from jax.experimental.pallas import tpu_sc as plsc

_N = 2 * 96 * 224 * 224
_NW = 32
_CH = 21504
_NCH = 6
_PER_W = _CH * _NCH
_N_SC = _NW * _PER_W
_N_TC = _N - _N_SC
_L = 16
_UN = 8

_THRESH = [0.75, 0.375, 0.1875, 0.09375, 0.046875, 0.0234375, 0.01171875]
_VALS = [1.0, 0.5, 0.25, 0.125, 0.0625, 0.03125, 0.015625, 0.0078125]


def _quant_vec(v):
    a = jnp.abs(v)
    mag = jnp.full_like(a, _VALS[7])
    for t, val in zip(reversed(_THRESH), reversed(_VALS[:7])):
        mag = jnp.where(a >= t, val, mag)
    neg = v <= 0.0
    return jnp.where(neg, -mag, mag)


def _compute_chunk(in_b, out_b):
    def fb(j, c):
        o = j * (_L * _UN)
        for u in range(_UN):
            s = pl.ds(o + u * _L, _L)
            out_b[s] = _quant_vec(in_b[s])
        return c

    lax.fori_loop(0, _CH // (_L * _UN), fb, jnp.int32(0))


def _sc_body(x_hbm, o_hbm, in0, in1, out0, out1, si0, si1, so0, so1):
    wid = lax.axis_index("s") * 2 + lax.axis_index("c")
    base = wid * _PER_W
    bufs_in = (in0, in1)
    bufs_out = (out0, out1)
    sems_in = (si0, si1)
    sems_out = (so0, so1)
    in_h = [None, None]
    out_h = [None, None]
    in_h[0] = pltpu.async_copy(x_hbm.at[pl.ds(base, _CH)], bufs_in[0],
                               sems_in[0])
    for i in range(_NCH):
        b = i % 2
        nb = (i + 1) % 2
        if i + 1 < _NCH:
            in_h[nb] = pltpu.async_copy(
                x_hbm.at[pl.ds(base + (i + 1) * _CH, _CH)], bufs_in[nb],
                sems_in[nb])
        in_h[b].wait()
        if i >= 2:
            out_h[b].wait()
        _compute_chunk(bufs_in[b], bufs_out[b])
        out_h[b] = pltpu.async_copy(bufs_out[b],
                                    o_hbm.at[pl.ds(base + i * _CH, _CH)],
                                    sems_out[b])
    out_h[(_NCH - 2) % 2].wait()
    out_h[(_NCH - 1) % 2].wait()


def _sc_kernel(xf):
    mesh = plsc.VectorSubcoreMesh(core_axis_name="c", subcore_axis_name="s")
    run = pl.kernel(
        _sc_body,
        out_type=jax.ShapeDtypeStruct((_N_SC,), jnp.float32),
        mesh=mesh,
        scratch_types=[
            pltpu.VMEM((_CH,), jnp.float32), pltpu.VMEM((_CH,), jnp.float32),
            pltpu.VMEM((_CH,), jnp.float32), pltpu.VMEM((_CH,), jnp.float32),
            pltpu.SemaphoreType.DMA, pltpu.SemaphoreType.DMA,
            pltpu.SemaphoreType.DMA, pltpu.SemaphoreType.DMA,
        ],
    )
    return run(xf)


def _quant_tc_block(x):
    a = jnp.clip(jnp.abs(x), 0.0078125, 1.0)
    bits = lax.bitcast_convert_type(a, jnp.int32)
    neg = x <= 0.0
    add = jnp.where(neg, jnp.int32(0x400000), jnp.int32(0x3FFFFF))
    pb = (bits + add) & jnp.int32(0x7F800000)
    mag = lax.bitcast_convert_type(pb, jnp.float32)
    return jnp.where(neg, -mag, mag)


def _tc_body(x_ref, o_ref):
    o_ref[...] = _quant_tc_block(x_ref[...])


_COLS = 1024
_TC_ROWS = _N_TC // _COLS
_SC_ROWS = _N_SC // _COLS
_TC_BLK = 224
_TC_GRID = _TC_ROWS // _TC_BLK
_SC_ROW_OFF = _SC_ROWS // _TC_BLK


def _tc_kernel(xf2d):
    return pl.pallas_call(
        _tc_body,
        out_shape=jax.ShapeDtypeStruct((_TC_ROWS, _COLS), jnp.float32),
        grid=(_TC_GRID,),
        in_specs=[pl.BlockSpec((_TC_BLK, _COLS),
                               lambda i: (i + _SC_ROW_OFF, 0))],
        out_specs=pl.BlockSpec((_TC_BLK, _COLS), lambda i: (i, 0)),
    )(xf2d)


_NAT_ROWS = 2 * 96 * 224
_NAT_COLS = 224
_NAT_BLK = 7168
_NAT_GRID = _NAT_ROWS // _NAT_BLK


def _tc_native(x3):
    return pl.pallas_call(
        _tc_body,
        out_shape=jax.ShapeDtypeStruct((_NAT_ROWS, _NAT_COLS), jnp.float32),
        grid=(_NAT_GRID,),
        in_specs=[pl.BlockSpec((_NAT_BLK, _NAT_COLS), lambda i: (i, 0))],
        out_specs=pl.BlockSpec((_NAT_BLK, _NAT_COLS), lambda i: (i, 0)),
    )(x3)


def kernel(x, pow2_values):
    B, C, W, H = x.shape
    out = _tc_native(x.reshape(_NAT_ROWS, _NAT_COLS))
    return out.reshape(B, C, W, H)


# speedup vs baseline: 8.8175x; 1.0111x over previous
import jax
import jax.numpy as jnp
from jax import lax
from jax.experimental import pallas as pl
from jax.experimental.pallas import tpu as pltpu
from jax.experimental.pallas import tpu_sc as plsc

_N = 2 * 96 * 224 * 224
_NW = 32
_CH = 21504
_NCH = 6
_PER_W = _CH * _NCH
_N_SC = _NW * _PER_W
_N_TC = _N - _N_SC
_L = 16
_UN = 8

_THRESH = [0.75, 0.375, 0.1875, 0.09375, 0.046875, 0.0234375, 0.01171875]
_VALS = [1.0, 0.5, 0.25, 0.125, 0.0625, 0.03125, 0.015625, 0.0078125]


def _quant_vec(v):
    a = jnp.abs(v)
    mag = jnp.full_like(a, _VALS[7])
    for t, val in zip(reversed(_THRESH), reversed(_VALS[:7])):
        mag = jnp.where(a >= t, val, mag)
    neg = v <= 0.0
    return jnp.where(neg, -mag, mag)


def _compute_chunk(in_b, out_b):
    def fb(j, c):
        o = j * (_L * _UN)
        for u in range(_UN):
            s = pl.ds(o + u * _L, _L)
            out_b[s] = _quant_vec(in_b[s])
        return c

    lax.fori_loop(0, _CH // (_L * _UN), fb, jnp.int32(0))


def _sc_body(x_hbm, o_hbm, in0, in1, out0, out1, si0, si1, so0, so1):
    wid = lax.axis_index("s") * 2 + lax.axis_index("c")
    base = wid * _PER_W
    bufs_in = (in0, in1)
    bufs_out = (out0, out1)
    sems_in = (si0, si1)
    sems_out = (so0, so1)
    in_h = [None, None]
    out_h = [None, None]
    in_h[0] = pltpu.async_copy(x_hbm.at[pl.ds(base, _CH)], bufs_in[0],
                               sems_in[0])
    for i in range(_NCH):
        b = i % 2
        nb = (i + 1) % 2
        if i + 1 < _NCH:
            in_h[nb] = pltpu.async_copy(
                x_hbm.at[pl.ds(base + (i + 1) * _CH, _CH)], bufs_in[nb],
                sems_in[nb])
        in_h[b].wait()
        if i >= 2:
            out_h[b].wait()
        _compute_chunk(bufs_in[b], bufs_out[b])
        out_h[b] = pltpu.async_copy(bufs_out[b],
                                    o_hbm.at[pl.ds(base + i * _CH, _CH)],
                                    sems_out[b])
    out_h[(_NCH - 2) % 2].wait()
    out_h[(_NCH - 1) % 2].wait()


def _sc_kernel(xf):
    mesh = plsc.VectorSubcoreMesh(core_axis_name="c", subcore_axis_name="s")
    run = pl.kernel(
        _sc_body,
        out_type=jax.ShapeDtypeStruct((_N_SC,), jnp.float32),
        mesh=mesh,
        scratch_types=[
            pltpu.VMEM((_CH,), jnp.float32), pltpu.VMEM((_CH,), jnp.float32),
            pltpu.VMEM((_CH,), jnp.float32), pltpu.VMEM((_CH,), jnp.float32),
            pltpu.SemaphoreType.DMA, pltpu.SemaphoreType.DMA,
            pltpu.SemaphoreType.DMA, pltpu.SemaphoreType.DMA,
        ],
    )
    return run(xf)


def _quant_tc_block(x):
    a = jnp.clip(jnp.abs(x), 0.0078125, 1.0)
    bits = lax.bitcast_convert_type(a, jnp.int32)
    neg = x <= 0.0
    add = jnp.where(neg, jnp.int32(0x400000), jnp.int32(0x3FFFFF))
    pb = (bits + add) & jnp.int32(0x7F800000)
    mag = lax.bitcast_convert_type(pb, jnp.float32)
    return jnp.where(neg, -mag, mag)


def _tc_body(x_ref, o_ref):
    o_ref[...] = _quant_tc_block(x_ref[...])


_COLS = 1024
_TC_ROWS = _N_TC // _COLS
_SC_ROWS = _N_SC // _COLS
_TC_BLK = 224
_TC_GRID = _TC_ROWS // _TC_BLK
_SC_ROW_OFF = _SC_ROWS // _TC_BLK


def _tc_kernel(xf2d):
    return pl.pallas_call(
        _tc_body,
        out_shape=jax.ShapeDtypeStruct((_TC_ROWS, _COLS), jnp.float32),
        grid=(_TC_GRID,),
        in_specs=[pl.BlockSpec((_TC_BLK, _COLS),
                               lambda i: (i + _SC_ROW_OFF, 0))],
        out_specs=pl.BlockSpec((_TC_BLK, _COLS), lambda i: (i, 0)),
    )(xf2d)


_NAT_ROWS = 2 * 96 * 224
_NAT_COLS = 224
_NAT_BLK = 10752
_NAT_GRID = _NAT_ROWS // _NAT_BLK


def _tc_native(x3):
    return pl.pallas_call(
        _tc_body,
        out_shape=jax.ShapeDtypeStruct((_NAT_ROWS, _NAT_COLS), jnp.float32),
        grid=(_NAT_GRID,),
        in_specs=[pl.BlockSpec((_NAT_BLK, _NAT_COLS), lambda i: (i, 0))],
        out_specs=pl.BlockSpec((_NAT_BLK, _NAT_COLS), lambda i: (i, 0)),
    )(x3)


def kernel(x, pow2_values):
    B, C, W, H = x.shape
    out = _tc_native(x.reshape(_NAT_ROWS, _NAT_COLS))
    return out.reshape(B, C, W, H)


# speedup vs baseline: 8.8353x; 1.0020x over previous
import jax
import jax.numpy as jnp
from jax import lax
from jax.experimental import pallas as pl
from jax.experimental.pallas import tpu as pltpu
from jax.experimental.pallas import tpu_sc as plsc


_THRESH = [0.75, 0.375, 0.1875, 0.09375, 0.046875, 0.0234375, 0.01171875]
_VALS = [1.0, 0.5, 0.25, 0.125, 0.0625, 0.03125, 0.015625, 0.0078125]


def _quant_tc_block(x):
    a = jnp.clip(jnp.abs(x), 0.0078125, 1.0)
    bits = lax.bitcast_convert_type(a, jnp.int32)
    neg = x <= 0.0
    add = jnp.where(neg, jnp.int32(0x400000), jnp.int32(0x3FFFFF))
    pb = (bits + add) & jnp.int32(0x7F800000)
    mag = lax.bitcast_convert_type(pb, jnp.float32)
    return jnp.where(neg, -mag, mag)


def _tc_body(x_ref, o_ref):
    o_ref[...] = _quant_tc_block(x_ref[...])


_NAT_ROWS = 2 * 96 * 224
_NAT_COLS = 224
_NAT_BLK = 10752
_NAT_GRID = _NAT_ROWS // _NAT_BLK


def _tc_native(x2d):
    return pl.pallas_call(
        _tc_body,
        out_shape=jax.ShapeDtypeStruct((_NAT_ROWS, _NAT_COLS), jnp.float32),
        grid=(_NAT_GRID,),
        in_specs=[pl.BlockSpec((_NAT_BLK, _NAT_COLS), lambda i: (i, 0))],
        out_specs=pl.BlockSpec((_NAT_BLK, _NAT_COLS), lambda i: (i, 0)),
    )(x2d)


def kernel(x, pow2_values):
    B, C, W, H = x.shape
    out = _tc_native(x.reshape(_NAT_ROWS, _NAT_COLS))
    return out.reshape(B, C, W, H)


_N = 2 * 96 * 224 * 224
_NW = 32
_PER_W = _N // _NW
_CH = 21504
_NCH = _PER_W // _CH
_L = 16
_UN = 8


def _quant_vec(v):
    a = jnp.abs(v)
    mag = jnp.full_like(a, _VALS[7])
    for t, val in zip(reversed(_THRESH), reversed(_VALS[:7])):
        mag = jnp.where(a >= t, val, mag)
    neg = v <= 0.0
    return jnp.where(neg, -mag, mag)


def _compute_chunk(in_b, out_b):
    def fb(j, c):
        o = j * (_L * _UN)
        for u in range(_UN):
            s = pl.ds(o + u * _L, _L)
            out_b[s] = _quant_vec(in_b[s])
        return c

    lax.fori_loop(0, _CH // (_L * _UN), fb, jnp.int32(0))


def _sc_body(x_hbm, o_hbm, in0, in1, out0, out1, si0, si1, so0, so1):
    wid = lax.axis_index("s") * 2 + lax.axis_index("c")
    base = wid * _PER_W
    bufs_in = (in0, in1)
    bufs_out = (out0, out1)
    sems_in = (si0, si1)
    sems_out = (so0, so1)
    in_h = [None, None]
    out_h = [None, None]
    in_h[0] = pltpu.async_copy(x_hbm.at[pl.ds(base, _CH)], bufs_in[0],
                               sems_in[0])
    for i in range(_NCH):
        b = i % 2
        nb = (i + 1) % 2
        if i + 1 < _NCH:
            in_h[nb] = pltpu.async_copy(
                x_hbm.at[pl.ds(base + (i + 1) * _CH, _CH)], bufs_in[nb],
                sems_in[nb])
        in_h[b].wait()
        if i >= 2:
            out_h[b].wait()
        _compute_chunk(bufs_in[b], bufs_out[b])
        out_h[b] = pltpu.async_copy(bufs_out[b],
                                    o_hbm.at[pl.ds(base + i * _CH, _CH)],
                                    sems_out[b])
    out_h[(_NCH - 2) % 2].wait()
    out_h[(_NCH - 1) % 2].wait()


def _sc_kernel(xf):
    mesh = plsc.VectorSubcoreMesh(core_axis_name="c", subcore_axis_name="s")
    run = pl.kernel(
        _sc_body,
        out_type=jax.ShapeDtypeStruct((_N,), jnp.float32),
        mesh=mesh,
        scratch_types=[
            pltpu.VMEM((_CH,), jnp.float32), pltpu.VMEM((_CH,), jnp.float32),
            pltpu.VMEM((_CH,), jnp.float32), pltpu.VMEM((_CH,), jnp.float32),
            pltpu.SemaphoreType.DMA, pltpu.SemaphoreType.DMA,
            pltpu.SemaphoreType.DMA, pltpu.SemaphoreType.DMA,
        ],
    )
    return run(xf)
